# Initial kernel scaffold; baseline (speedup 1.0000x reference)
import jax
import jax.numpy as jnp
from jax.experimental import pallas as pl


def kernel(embedding, centroids_0, centroids_1, centroids_2):
    raise NotImplementedError("write your pallas kernel here")


# speedup vs baseline: 42.8479x; 42.8479x over previous
import functools

import jax
import jax.numpy as jnp
from jax import lax
from jax.experimental import pallas as pl
from jax.experimental.pallas import tpu as pltpu
from jax.experimental.pallas import tpu_sc as plsc

BATCH = 4096
D = 256
K = 64
N1 = K * K
CHUNK = 512
NTILES = 32
PER = BATCH // NTILES
LANES = 16


def _tc01_body(e_ref, c0_ref, c1_ref, p0_ref, d0_ref, p1_ref, d1_ref, ridx_ref):
    e = e_ref[...]

    def l0_body(j, carry):
        best, bestj = carry
        c0j = c0_ref[pl.ds(j, 1), :]
        n = jnp.sqrt(jnp.sum((e - c0j) ** 2, axis=1))
        upd = n < best
        return jnp.where(upd, n, best), jnp.where(upd, j, bestj)

    best0 = jnp.full((CHUNK,), jnp.inf, jnp.float32)
    bestj0 = jnp.zeros((CHUNK,), jnp.int32)
    best0, bestj0 = lax.fori_loop(0, K, l0_body, (best0, bestj0))
    p0_ref[...] = bestj0
    d0_ref[...] = best0

    lo = bestj0 * K
    norme = jnp.sum(e * e, axis=1)
    lhs = jnp.concatenate([-2.0 * e, jnp.ones((CHUNK, 1), jnp.float32)], axis=1)
    CC = 512

    def l1_body(q, carry):
        best, bestj = carry
        c1q = c1_ref[pl.ds(q * CC, CC), :]
        normq = jnp.sum(c1q * c1q, axis=1)
        rhs = jnp.concatenate([c1q, normq[:, None]], axis=1)
        scores = lax.dot_general(lhs, rhs, (((1,), (1,)), ((), ())),
                                 preferred_element_type=jnp.float32,
                                 precision=lax.Precision.HIGHEST)
        col = q * CC + lax.broadcasted_iota(jnp.int32, (CHUNK, CC), 1)
        inwin = (col >= lo[:, None]) & (col < lo[:, None] + K)
        masked = jnp.where(inwin, scores, jnp.inf)
        m = jnp.min(masked, axis=1)
        am = jnp.argmin(masked, axis=1).astype(jnp.int32) + q * CC
        upd = m < best
        return jnp.where(upd, m, best), jnp.where(upd, am, bestj)

    best1 = jnp.full((CHUNK,), jnp.inf, jnp.float32)
    bestc1 = jnp.zeros((CHUNK,), jnp.int32)
    best1, p1 = lax.fori_loop(0, N1 // CC, l1_body, (best1, bestc1))
    d1 = jnp.sqrt(jnp.maximum(best1 + norme, 0.0))
    p1_ref[...] = p1
    d1_ref[...] = d1
    ridx_ref[...] = p1[:, None] * K + lax.broadcasted_iota(jnp.int32, (CHUNK, K), 1)


_tc01 = pl.pallas_call(
    _tc01_body,
    grid=(BATCH // CHUNK,),
    in_specs=[
        pl.BlockSpec((CHUNK, D), lambda i: (i, 0)),
        pl.BlockSpec((K, D), lambda i: (0, 0)),
        pl.BlockSpec((N1, D), lambda i: (0, 0)),
    ],
    out_specs=[
        pl.BlockSpec((CHUNK,), lambda i: (i,)),
        pl.BlockSpec((CHUNK,), lambda i: (i,)),
        pl.BlockSpec((CHUNK,), lambda i: (i,)),
        pl.BlockSpec((CHUNK,), lambda i: (i,)),
        pl.BlockSpec((CHUNK, K), lambda i: (i, 0)),
    ],
    out_shape=[
        jax.ShapeDtypeStruct((BATCH,), jnp.int32),
        jax.ShapeDtypeStruct((BATCH,), jnp.float32),
        jax.ShapeDtypeStruct((BATCH,), jnp.int32),
        jax.ShapeDtypeStruct((BATCH,), jnp.float32),
        jax.ShapeDtypeStruct((BATCH, K), jnp.int32),
    ],
)


def _sc2_body(c2_hbm, emb_hbm, ridx_hbm, oi_hbm, od_hbm,
              idx_v, emb_v, rows_v, part_v, oi_v, od_v, sem0, sem1):
    cid = lax.axis_index("c")
    sid = lax.axis_index("s")
    wid = sid * 2 + cid
    base = wid * PER
    pltpu.sync_copy(ridx_hbm.at[pl.ds(base, PER)], idx_v)
    pltpu.sync_copy(emb_hbm.at[pl.ds(base, PER)], emb_v)

    sems = (sem0, sem1)
    lanes = lax.iota(jnp.int32, LANES)
    onelane = lanes == 0
    zeros16 = jnp.zeros((LANES,), jnp.int32)

    pltpu.async_copy(c2_hbm.at[idx_v.at[0]], rows_v.at[0], sem0)

    def do_embedding(b, p):
        pltpu.make_async_copy(c2_hbm.at[pl.ds(0, K)], rows_v.at[p], sems[p]).wait()

        @pl.when(b + 1 < PER)
        def _():
            pltpu.async_copy(c2_hbm.at[idx_v.at[b + 1]], rows_v.at[1 - p],
                             sems[1 - p])

        rows = rows_v.at[p]
        ev = [emb_v[b, pl.ds(LANES * c, LANES)] for c in range(D // LANES)]

        best = jnp.full((LANES,), jnp.inf, jnp.float32)
        bestj = zeros16
        for g in range(K // LANES):
            def row_body(r, carry):
                acc = jnp.zeros((LANES,), jnp.float32)
                j = g * LANES + r
                for c in range(D // LANES):
                    dd = rows[j, pl.ds(LANES * c, LANES)] - ev[c]
                    acc = dd * dd + acc
                part_v[pl.ds(r * LANES, LANES)] = acc
                return carry

            lax.fori_loop(0, LANES, row_body, jnp.asarray(0, jnp.int32))
            tot = jnp.zeros((LANES,), jnp.float32)
            for t in range(LANES):
                tot = tot + plsc.load_gather(part_v, [lanes * LANES + t])
            jidx = lanes + g * LANES
            upd = tot < best
            best = jnp.where(upd, tot, best)
            bestj = jnp.where(upd, jidx, bestj)

        sk, sv = plsc.sort_key_val(best, bestj)
        bvec = jnp.full((LANES,), b, jnp.int32)
        winbase = plsc.load_gather(idx_v, [bvec, zeros16])
        plsc.store_scatter(oi_v, [bvec], winbase + sv, mask=onelane)
        plsc.store_scatter(od_v, [bvec], sk, mask=onelane)

    def loop_body(i, carry):
        for p in range(2):
            do_embedding(2 * i + p, p)
        return carry

    lax.fori_loop(0, PER // 2, loop_body, jnp.asarray(0, jnp.int32))
    pltpu.sync_copy(oi_v, oi_hbm.at[pl.ds(base, PER)])
    pltpu.sync_copy(od_v, od_hbm.at[pl.ds(base, PER)])


@functools.lru_cache(maxsize=None)
def _make_sc2():
    return functools.partial(
        pl.kernel,
        out_type=(jax.ShapeDtypeStruct((BATCH,), jnp.int32),
                  jax.ShapeDtypeStruct((BATCH,), jnp.float32)),
        mesh=plsc.VectorSubcoreMesh(core_axis_name="c", subcore_axis_name="s"),
        compiler_params=pltpu.CompilerParams(needs_layout_passes=False),
        scratch_types=[
            pltpu.VMEM((PER, K), jnp.int32),
            pltpu.VMEM((PER, D), jnp.float32),
            pltpu.VMEM((2, K, D), jnp.float32),
            pltpu.VMEM((LANES * LANES,), jnp.float32),
            pltpu.VMEM((PER,), jnp.int32),
            pltpu.VMEM((PER,), jnp.float32),
            pltpu.SemaphoreType.DMA,
            pltpu.SemaphoreType.DMA,
        ],
    )(_sc2_body)


def kernel(embedding, centroids_0, centroids_1, centroids_2):
    p0, d0, p1, d1, ridx = _tc01(embedding, centroids_0, centroids_1)
    leaf, d2sq = _make_sc2()(centroids_2, embedding, ridx)
    d2 = jnp.sqrt(d2sq)
    paths = jnp.stack([p0, p1, leaf], axis=1)
    dists = jnp.stack([d0, d1, d2], axis=1)
    return leaf, paths, dists


# speedup vs baseline: 51.3435x; 1.1983x over previous
import functools

import jax
import jax.numpy as jnp
from jax import lax
from jax.experimental import pallas as pl
from jax.experimental.pallas import tpu as pltpu
from jax.experimental.pallas import tpu_sc as plsc

BATCH = 4096
D = 256
K = 64
N1 = K * K
CHUNK = 512
NTILES = 32
PER = BATCH // NTILES
LANES = 16


def _tc01_body(e_ref, c0_ref, c1_ref, p0_ref, d0_ref, p1_ref, d1_ref, ridx_ref):
    e = e_ref[...]
    norme = jnp.sum(e * e, axis=1)
    lhs = jnp.concatenate([-2.0 * e, jnp.ones((CHUNK, 1), jnp.float32)], axis=1)

    c0 = c0_ref[...]
    normc0 = jnp.sum(c0 * c0, axis=1)
    rhs0 = jnp.concatenate([c0, normc0[:, None]], axis=1)
    scores0 = lax.dot_general(lhs, rhs0, (((1,), (1,)), ((), ())),
                              preferred_element_type=jnp.float32,
                              precision=lax.Precision.HIGHEST)
    col64 = lax.broadcasted_iota(jnp.int32, (CHUNK, K), 1)
    best0 = jnp.full((CHUNK,), jnp.inf, jnp.float32)
    bestj0 = jnp.zeros((CHUNK,), jnp.int32)
    cur = scores0
    for k in range(4):
        jk = jnp.argmin(cur, axis=1).astype(jnp.int32)
        sel = col64 == jk[:, None]
        oh = sel.astype(jnp.float32)
        ck = lax.dot_general(oh, c0, (((1,), (0,)), ((), ())),
                             preferred_element_type=jnp.float32,
                             precision=lax.Precision.HIGHEST)
        nk = jnp.sqrt(jnp.sum((e - ck) ** 2, axis=1))
        upd = (nk < best0) | ((nk == best0) & (jk < bestj0))
        best0 = jnp.where(upd, nk, best0)
        bestj0 = jnp.where(upd, jk, bestj0)
        if k < 3:
            cur = jnp.where(sel, jnp.inf, cur)
    p0_ref[...] = bestj0
    d0_ref[...] = best0

    lo = bestj0 * K
    norme = jnp.sum(e * e, axis=1)
    lhs = jnp.concatenate([-2.0 * e, jnp.ones((CHUNK, 1), jnp.float32)], axis=1)
    CC = 512

    def l1_body(q, carry):
        best, bestj = carry
        c1q = c1_ref[pl.ds(q * CC, CC), :]
        normq = jnp.sum(c1q * c1q, axis=1)
        rhs = jnp.concatenate([c1q, normq[:, None]], axis=1)
        scores = lax.dot_general(lhs, rhs, (((1,), (1,)), ((), ())),
                                 preferred_element_type=jnp.float32,
                                 precision=lax.Precision.HIGHEST)
        col = q * CC + lax.broadcasted_iota(jnp.int32, (CHUNK, CC), 1)
        inwin = (col >= lo[:, None]) & (col < lo[:, None] + K)
        masked = jnp.where(inwin, scores, jnp.inf)
        m = jnp.min(masked, axis=1)
        am = jnp.argmin(masked, axis=1).astype(jnp.int32) + q * CC
        upd = m < best
        return jnp.where(upd, m, best), jnp.where(upd, am, bestj)

    best1 = jnp.full((CHUNK,), jnp.inf, jnp.float32)
    bestc1 = jnp.zeros((CHUNK,), jnp.int32)
    best1, p1 = lax.fori_loop(0, N1 // CC, l1_body, (best1, bestc1))
    d1 = jnp.sqrt(jnp.maximum(best1 + norme, 0.0))
    p1_ref[...] = p1
    d1_ref[...] = d1
    ridx_ref[...] = p1[:, None] * K + lax.broadcasted_iota(jnp.int32, (CHUNK, K), 1)


_tc01 = pl.pallas_call(
    _tc01_body,
    grid=(BATCH // CHUNK,),
    in_specs=[
        pl.BlockSpec((CHUNK, D), lambda i: (i, 0)),
        pl.BlockSpec((K, D), lambda i: (0, 0)),
        pl.BlockSpec((N1, D), lambda i: (0, 0)),
    ],
    out_specs=[
        pl.BlockSpec((CHUNK,), lambda i: (i,)),
        pl.BlockSpec((CHUNK,), lambda i: (i,)),
        pl.BlockSpec((CHUNK,), lambda i: (i,)),
        pl.BlockSpec((CHUNK,), lambda i: (i,)),
        pl.BlockSpec((CHUNK, K), lambda i: (i, 0)),
    ],
    out_shape=[
        jax.ShapeDtypeStruct((BATCH,), jnp.int32),
        jax.ShapeDtypeStruct((BATCH,), jnp.float32),
        jax.ShapeDtypeStruct((BATCH,), jnp.int32),
        jax.ShapeDtypeStruct((BATCH,), jnp.float32),
        jax.ShapeDtypeStruct((BATCH, K), jnp.int32),
    ],
)


def _sc2_body(c2_hbm, emb_hbm, ridx_hbm, oi_hbm, od_hbm,
              idx_v, emb_v, rows_v, part_v, oi_v, od_v, sem0, sem1):
    cid = lax.axis_index("c")
    sid = lax.axis_index("s")
    wid = sid * 2 + cid
    base = wid * PER
    pltpu.sync_copy(ridx_hbm.at[pl.ds(base, PER)], idx_v)
    pltpu.sync_copy(emb_hbm.at[pl.ds(base, PER)], emb_v)

    sems = (sem0, sem1)
    lanes = lax.iota(jnp.int32, LANES)
    onelane = lanes == 0
    zeros16 = jnp.zeros((LANES,), jnp.int32)

    pltpu.async_copy(c2_hbm.at[idx_v.at[0]], rows_v.at[0], sem0)

    def do_embedding(b, p):
        pltpu.make_async_copy(c2_hbm.at[pl.ds(0, K)], rows_v.at[p], sems[p]).wait()

        @pl.when(b + 1 < PER)
        def _():
            pltpu.async_copy(c2_hbm.at[idx_v.at[b + 1]], rows_v.at[1 - p],
                             sems[1 - p])

        rows = rows_v.at[p]
        ev = [emb_v[b, pl.ds(LANES * c, LANES)] for c in range(D // LANES)]

        best = jnp.full((LANES,), jnp.inf, jnp.float32)
        bestj = zeros16
        for g in range(K // LANES):
            def row_body(r, carry):
                acc = jnp.zeros((LANES,), jnp.float32)
                j = g * LANES + r
                for c in range(D // LANES):
                    dd = rows[j, pl.ds(LANES * c, LANES)] - ev[c]
                    acc = dd * dd + acc
                part_v[pl.ds(r * LANES, LANES)] = acc
                return carry

            lax.fori_loop(0, LANES, row_body, jnp.asarray(0, jnp.int32))
            tot = jnp.zeros((LANES,), jnp.float32)
            for t in range(LANES):
                tot = tot + plsc.load_gather(part_v, [lanes * LANES + t])
            jidx = lanes + g * LANES
            upd = tot < best
            best = jnp.where(upd, tot, best)
            bestj = jnp.where(upd, jidx, bestj)

        sk, sv = plsc.sort_key_val(best, bestj)
        bvec = jnp.full((LANES,), b, jnp.int32)
        winbase = plsc.load_gather(idx_v, [bvec, zeros16])
        plsc.store_scatter(oi_v, [bvec], winbase + sv, mask=onelane)
        plsc.store_scatter(od_v, [bvec], sk, mask=onelane)

    def loop_body(i, carry):
        for p in range(2):
            do_embedding(2 * i + p, p)
        return carry

    lax.fori_loop(0, PER // 2, loop_body, jnp.asarray(0, jnp.int32))
    pltpu.sync_copy(oi_v, oi_hbm.at[pl.ds(base, PER)])
    pltpu.sync_copy(od_v, od_hbm.at[pl.ds(base, PER)])


@functools.lru_cache(maxsize=None)
def _make_sc2():
    return functools.partial(
        pl.kernel,
        out_type=(jax.ShapeDtypeStruct((BATCH,), jnp.int32),
                  jax.ShapeDtypeStruct((BATCH,), jnp.float32)),
        mesh=plsc.VectorSubcoreMesh(core_axis_name="c", subcore_axis_name="s"),
        compiler_params=pltpu.CompilerParams(needs_layout_passes=False),
        scratch_types=[
            pltpu.VMEM((PER, K), jnp.int32),
            pltpu.VMEM((PER, D), jnp.float32),
            pltpu.VMEM((2, K, D), jnp.float32),
            pltpu.VMEM((LANES * LANES,), jnp.float32),
            pltpu.VMEM((PER,), jnp.int32),
            pltpu.VMEM((PER,), jnp.float32),
            pltpu.SemaphoreType.DMA,
            pltpu.SemaphoreType.DMA,
        ],
    )(_sc2_body)


def kernel(embedding, centroids_0, centroids_1, centroids_2):
    p0, d0, p1, d1, ridx = _tc01(embedding, centroids_0, centroids_1)
    leaf, d2sq = _make_sc2()(centroids_2, embedding, ridx)
    d2 = jnp.sqrt(d2sq)
    paths = jnp.stack([p0, p1, leaf], axis=1)
    dists = jnp.stack([d0, d1, d2], axis=1)
    return leaf, paths, dists
